# Initial kernel scaffold; baseline (speedup 1.0000x reference)
import jax
import jax.numpy as jnp
from jax.experimental import pallas as pl


def kernel(object_features, language_embedding, centers, sizes, W1, b1, W2, b2):
    raise NotImplementedError("write your pallas kernel here")


# speedup vs baseline: 21.2974x; 21.2974x over previous
import jax
import jax.numpy as jnp
from jax.experimental import pallas as pl


def _proj_kernel(feats_ref, lang_ref, centers_ref, sizes_ref,
                 w1a_ref, w1b_ref, w1gp_ref, w1gs_ref, w1l_ref, b1_ref,
                 base2_ref, g2_ref):
    f = feats_ref[0]
    fa = jnp.dot(f, w1a_ref[...], preferred_element_type=jnp.float32)
    fb = jnp.dot(f, w1b_ref[...], preferred_element_type=jnp.float32)
    c = centers_ref[0]
    s = sizes_ref[0]
    cs = c[:, 0:1] * w1gp_ref[0:1, :] + s[:, 0:1] * w1gs_ref[0:1, :]
    for dd in range(1, 3):
        cs = cs + c[:, dd:dd + 1] * w1gp_ref[dd:dd + 1, :]
        cs = cs + s[:, dd:dd + 1] * w1gs_ref[dd:dd + 1, :]
    lb = jnp.dot(lang_ref[0], w1l_ref[...],
                 preferred_element_type=jnp.float32)
    base2_ref[0] = fa + cs + (lb + b1_ref[...])
    g2_ref[0] = fb - cs


def _main_kernel(centers_ref, centersT_ref, base2_ref, g2_ref, feats_ref,
                 w2t_ref, enh_ref, w_ref, idx_ref):
    T = centers_ref.shape[1]
    N = g2_ref.shape[1]
    K = idx_ref.shape[2]
    r0 = pl.program_id(1) * T

    cb = centers_ref[0]
    acc = None
    for dd in range(3):
        diff = cb[:, dd:dd + 1] - centersT_ref[0, dd:dd + 1, :]
        sq = diff * diff
        acc = sq if acc is None else acc + sq
    col = jax.lax.broadcasted_iota(jnp.int32, (T, N), 1)
    row_g = r0 + jax.lax.broadcasted_iota(jnp.int32, (T, N), 0)
    dist = jnp.where(col == row_g, jnp.inf, acc)

    idxs = []
    d = dist
    for _ in range(K):
        m = jnp.min(d, axis=1, keepdims=True)
        cand = jnp.where(d == m, col, N)
        aj = jnp.min(cand, axis=1, keepdims=True)
        idxs.append(aj)
        d = jnp.where(col == aj, jnp.inf, d)

    g2 = g2_ref[0]
    base2 = base2_ref[0]
    w2t = w2t_ref[...]
    ohs, scores = [], []
    for j in range(K):
        oh = (col == idxs[j]).astype(jnp.float32)
        ohs.append(oh)
        gj = jnp.dot(oh, g2, preferred_element_type=jnp.float32)
        h = jnp.maximum(base2 + gj, 0.0)
        scores.append(jnp.sum(h * w2t, axis=1, keepdims=True))

    s = jnp.concatenate(scores, axis=1)
    s = s - jnp.max(s, axis=1, keepdims=True)
    e = jnp.exp(s)
    w = e / jnp.sum(e, axis=1, keepdims=True)

    a = w[:, 0:1] * ohs[0]
    for j in range(1, K):
        a = a + w[:, j:j + 1] * ohs[j]
    feats = feats_ref[0]
    rc = jnp.dot(a, feats, preferred_element_type=jnp.float32)
    enh_ref[0] = feats_ref[0, pl.ds(r0, T), :] + rc
    w_ref[0] = w
    idx_ref[0] = jnp.concatenate(idxs, axis=1)


def kernel(object_features, language_embedding, centers, sizes, W1, b1, W2, b2):
    B, N, D = object_features.shape
    L = language_embedding.shape[1]
    H = b1.shape[0]
    K = min(5, N - 1)
    T = 256

    W1a = W1[:D]
    W1b = W1[D:2 * D]
    W1gp = W1[2 * D:2 * D + 3] / 5.0
    W1gs = W1[2 * D + 3:2 * D + 6] / 2.0
    W1l = W1[2 * D + 6:]
    b1r = b1.reshape(1, H)
    w2t = W2.reshape(1, H)
    lang3 = language_embedding.reshape(B, 1, L)
    centersT = jnp.swapaxes(centers, 1, 2)

    base2, g2 = pl.pallas_call(
        _proj_kernel,
        grid=(B,),
        in_specs=[
            pl.BlockSpec((1, N, D), lambda b: (b, 0, 0)),
            pl.BlockSpec((1, 1, L), lambda b: (b, 0, 0)),
            pl.BlockSpec((1, N, 3), lambda b: (b, 0, 0)),
            pl.BlockSpec((1, N, 3), lambda b: (b, 0, 0)),
            pl.BlockSpec((D, H), lambda b: (0, 0)),
            pl.BlockSpec((D, H), lambda b: (0, 0)),
            pl.BlockSpec((3, H), lambda b: (0, 0)),
            pl.BlockSpec((3, H), lambda b: (0, 0)),
            pl.BlockSpec((L, H), lambda b: (0, 0)),
            pl.BlockSpec((1, H), lambda b: (0, 0)),
        ],
        out_specs=[
            pl.BlockSpec((1, N, H), lambda b: (b, 0, 0)),
            pl.BlockSpec((1, N, H), lambda b: (b, 0, 0)),
        ],
        out_shape=[
            jax.ShapeDtypeStruct((B, N, H), jnp.float32),
            jax.ShapeDtypeStruct((B, N, H), jnp.float32),
        ],
    )(object_features, lang3, centers, sizes,
      W1a, W1b, W1gp, W1gs, W1l, b1r)

    enhanced, weights, nidx = pl.pallas_call(
        _main_kernel,
        grid=(B, N // T),
        in_specs=[
            pl.BlockSpec((1, T, 3), lambda b, t: (b, t, 0)),
            pl.BlockSpec((1, 3, N), lambda b, t: (b, 0, 0)),
            pl.BlockSpec((1, T, H), lambda b, t: (b, t, 0)),
            pl.BlockSpec((1, N, H), lambda b, t: (b, 0, 0)),
            pl.BlockSpec((1, N, D), lambda b, t: (b, 0, 0)),
            pl.BlockSpec((1, H), lambda b, t: (0, 0)),
        ],
        out_specs=[
            pl.BlockSpec((1, T, D), lambda b, t: (b, t, 0)),
            pl.BlockSpec((1, T, K), lambda b, t: (b, t, 0)),
            pl.BlockSpec((1, T, K), lambda b, t: (b, t, 0)),
        ],
        out_shape=[
            jax.ShapeDtypeStruct((B, N, D), jnp.float32),
            jax.ShapeDtypeStruct((B, N, K), jnp.float32),
            jax.ShapeDtypeStruct((B, N, K), jnp.int32),
        ],
    )(centers, centersT, base2, g2, object_features, w2t)

    return enhanced, weights, nidx
